# baseline (speedup 1.0000x reference)
import functools

import jax
import jax.numpy as jnp
from jax import lax
from jax.experimental import pallas as pl
from jax.experimental.pallas import tpu as pltpu
from jax.experimental.pallas import tpu_sc as plsc

_D = 4096


@functools.lru_cache(maxsize=None)
def _make_sc_lookup(B: int):
    info = plsc.get_sparse_core_info()
    nw = info.num_cores * info.num_subcores
    b_per_w = B // nw
    n_chunks = b_per_w // 16
    mesh = plsc.VectorSubcoreMesh(core_axis_name="c", subcore_axis_name="s")

    @functools.partial(
        pl.kernel,
        mesh=mesh,
        out_type=jax.ShapeDtypeStruct((B, _D), jnp.float32),
        scratch_types=[
            pltpu.VMEM((b_per_w,), jnp.int32),
            pltpu.VMEM((16, _D), jnp.float32),
            pltpu.SemaphoreType.DMA,
            pltpu.SemaphoreType.DMA,
        ],
    )
    def lookup(table_hbm, idx_hbm, ids_hbm, out_hbm, ids_v, src_v, gsem, psem):
        wid = lax.axis_index("s") * info.num_cores + lax.axis_index("c")
        base = wid * b_per_w
        pltpu.sync_copy(ids_hbm.at[pl.ds(base, b_per_w)], ids_v)
        zeros16 = jnp.zeros((16,), jnp.int32)
        pltpu.async_copy(table_hbm.at[zeros16], src_v, gsem).wait()

        def body(k, carry):
            v = ids_v[pl.ds(k * 16, 16)]
            pltpu.async_copy(src_v, out_hbm.at[v], psem).start()
            return carry

        lax.fori_loop(0, n_chunks, body, 0, unroll=False)

        def drain(k, carry):
            pltpu.make_async_copy(src_v, out_hbm.at[zeros16], psem).wait()
            return carry

        lax.fori_loop(0, n_chunks, drain, 0, unroll=False)

    return lookup


def kernel(x, emb_weight):
    b, s = x.shape
    idx = x.reshape(-1).astype(jnp.int32)
    ids = jnp.arange(b * s, dtype=jnp.int32)
    out = _make_sc_lookup(b * s)(emb_weight, idx, ids)
    return out.reshape(b, s, _D)


# speedup vs baseline: 1.0542x; 1.0542x over previous
import functools

import jax
import jax.numpy as jnp
from jax import lax
from jax.experimental import pallas as pl
from jax.experimental.pallas import tpu as pltpu
from jax.experimental.pallas import tpu_sc as plsc

_D = 4096


@functools.lru_cache(maxsize=None)
def _make_sc_lookup(B: int):
    info = plsc.get_sparse_core_info()
    nw = info.num_cores * info.num_subcores
    assert B % (8 * nw) == 0
    b_per_w = B // nw
    mesh = plsc.VectorSubcoreMesh(core_axis_name="c", subcore_axis_name="s")

    @functools.partial(
        pl.kernel,
        mesh=mesh,
        out_type=jax.ShapeDtypeStruct((B, _D), jnp.float32),
        scratch_types=[
            pltpu.VMEM((b_per_w,), jnp.int32),
            pltpu.VMEM((2, _D), jnp.float32),
            pltpu.SemaphoreType.DMA,
        ],
    )
    def lookup(table_hbm, idx_hbm, out_hbm, idx_v, w_v, psem):
        wid = lax.axis_index("s") * info.num_cores + lax.axis_index("c")
        base = wid * b_per_w
        pltpu.sync_copy(idx_hbm.at[pl.ds(base, b_per_w)], idx_v)
        pltpu.sync_copy(table_hbm, w_v)

        def body(k, carry):
            v16 = idx_v[pl.ds(k * 16, 16)]
            for j in range(16):
                s = jnp.squeeze(lax.slice(v16, (j,), (j + 1,)))
                pltpu.async_copy(
                    w_v.at[pl.ds(s, 1)],
                    out_hbm.at[pl.ds(base + k * 16 + j, 1)], psem
                ).start()
            return carry

        lax.fori_loop(0, b_per_w // 16, body, 0, unroll=False)

        def drain(i, carry):
            pltpu.make_async_copy(
                w_v.at[pl.ds(0, 1)], out_hbm.at[pl.ds(base, 1)], psem
            ).wait()
            return carry

        lax.fori_loop(0, b_per_w, drain, 0, unroll=False)

    return lookup


def kernel(x, emb_weight):
    b, s = x.shape
    idx = x.reshape(-1).astype(jnp.int32)
    out = _make_sc_lookup(b * s)(emb_weight, idx)
    return out.reshape(b, s, _D)


# speedup vs baseline: 1.2342x; 1.1708x over previous
import functools

import jax
import jax.numpy as jnp
from jax import lax
from jax.experimental import pallas as pl
from jax.experimental.pallas import tpu as pltpu
from jax.experimental.pallas import tpu_sc as plsc

_D = 4096
_C = 8
_NBUF = 2
_L = 16
_QUNROLL = 8


@functools.lru_cache(maxsize=None)
def _make_sc_lookup(B: int):
    info = plsc.get_sparse_core_info()
    nw = info.num_cores * info.num_subcores
    assert B % (8 * nw) == 0
    b_per_w = B // nw
    assert b_per_w % _C == 0
    n_chunks = b_per_w // _C
    assert n_chunks % _NBUF == 0 and n_chunks >= 2 * _NBUF
    n_q = _D // _L
    mesh = plsc.VectorSubcoreMesh(core_axis_name="c", subcore_axis_name="s")

    @functools.partial(
        pl.kernel,
        mesh=mesh,
        out_type=jax.ShapeDtypeStruct((B, _D), jnp.float32),
        scratch_types=(
            [pltpu.VMEM((b_per_w + _L,), jnp.int32),
             pltpu.VMEM((2, _D), jnp.float32)]
            + [pltpu.VMEM((_C, _D), jnp.float32)] * _NBUF
            + [pltpu.SemaphoreType.DMA] * _NBUF
        ),
    )
    def lookup(table_hbm, idx_hbm, out_hbm, idx_v, w_v, *bufs_sems):
        bufs = bufs_sems[:_NBUF]
        psems = bufs_sems[_NBUF:]
        wid = lax.axis_index("s") * info.num_cores + lax.axis_index("c")
        base = wid * b_per_w
        pltpu.sync_copy(idx_hbm.at[pl.ds(base, b_per_w)],
                        idx_v.at[pl.ds(0, b_per_w)])
        pltpu.sync_copy(table_hbm, w_v)

        def put_desc(c, p):
            return pltpu.make_async_copy(
                bufs[p], out_hbm.at[pl.ds(base + c * _C, _C)], psems[p])

        def step(c, p):
            @pl.when(c >= _NBUF)
            def _():
                put_desc(c - _NBUF, p).wait()

            v16 = idx_v[pl.ds(c * _C, _L)]
            m1s, m0s = [], []
            for r in range(_C):
                s = jnp.squeeze(lax.slice(v16, (r,), (r + 1,)))
                m1 = jnp.zeros((_L,), jnp.int32) - s
                m1s.append(m1)
                m0s.append(~m1)

            def build(q, carry):
                w0q = lax.bitcast_convert_type(
                    w_v[0, pl.ds(q * _L, _L)], jnp.int32)
                w1q = lax.bitcast_convert_type(
                    w_v[1, pl.ds(q * _L, _L)], jnp.int32)
                for r in range(_C):
                    sel = (w0q & m0s[r]) | (w1q & m1s[r])
                    bufs[p][r, pl.ds(q * _L, _L)] = lax.bitcast_convert_type(
                        sel, jnp.float32)
                return carry

            lax.fori_loop(0, n_q, build, 0, unroll=_QUNROLL)
            put_desc(c, p).start()

        def body(j2, carry):
            for p in range(_NBUF):
                step(j2 * _NBUF + p, p)
            return carry

        lax.fori_loop(0, n_chunks // _NBUF, body, 0, unroll=False)
        for k in range(_NBUF):
            put_desc(n_chunks - _NBUF + k, k).wait()

    return lookup


def kernel(x, emb_weight):
    b, s = x.shape
    idx = x.reshape(-1).astype(jnp.int32)
    out = _make_sc_lookup(b * s)(emb_weight, idx)
    return out.reshape(b, s, _D)
